# baseline (speedup 1.0000x reference)
import functools

import jax
import jax.numpy as jnp
from jax import lax
from jax.experimental import pallas as pl
from jax.experimental.pallas import tpu as pltpu
from jax.experimental.pallas import tpu_sc as plsc

F = 256
FH = 128
K = 2
NC = 2
NS = 16
EB = 64
SB = 8
BM = 512


def _round_up(a, b):
    return (a + b - 1) // b * b


def _make_deg(NP, EP):
    mesh = plsc.VectorSubcoreMesh(core_axis_name="c", subcore_axis_name="s")
    epw = EP // (NC * NS)
    nb = epw // EB

    @functools.partial(
        pl.kernel,
        mesh=mesh,
        out_type=jax.ShapeDtypeStruct((NC * NS, NP // 128, 128), jnp.float32),
        compiler_params=pltpu.CompilerParams(needs_layout_passes=False),
        scratch_types=[
            pltpu.VMEM((EB,), jnp.int32),
            pltpu.VMEM((EB,), jnp.float32),
            pltpu.VMEM((NP // 128, 128), jnp.float32),
        ],
    )
    def deg_kernel(col_hbm, ew_hbm, out_hbm, cidx_v, ew_v, deg_v):
        c = lax.axis_index("c")
        s = lax.axis_index("s")
        wid = s * NC + c
        lane = lax.iota(jnp.int32, 16)
        zero16 = jnp.zeros((16,), jnp.float32)

        def zero_body(r, carry):
            rf = jnp.full((16,), r, jnp.int32)
            for j in range(8):
                plsc.store_scatter(deg_v, [rf, j * 16 + lane], zero16)
            return carry

        lax.fori_loop(0, NP // 128, zero_body, 0)

        base0 = wid * epw

        def batch_body(b, carry):
            eb = pl.multiple_of(base0 + b * EB, EB)
            pltpu.sync_copy(col_hbm.at[pl.ds(eb, EB)], cidx_v)
            pltpu.sync_copy(ew_hbm.at[pl.ds(eb, EB)], ew_v)
            for j in range(EB // 16):
                idx = cidx_v[pl.ds(j * 16, 16)]
                w = ew_v[pl.ds(j * 16, 16)]
                plsc.addupdate_scatter(
                    deg_v, [lax.shift_right_logical(idx, 7), idx & 127], w)
            return carry

        lax.fori_loop(0, nb, batch_body, 0)
        pltpu.sync_copy(deg_v, out_hbm.at[wid])

    return deg_kernel


def _make_prop(NP, EP):
    mesh = plsc.VectorSubcoreMesh(core_axis_name="c", subcore_axis_name="s")
    ept = EP // NS
    nb = ept // EB
    rpt = NP // NS

    @functools.partial(
        pl.kernel,
        mesh=mesh,
        out_type=jax.ShapeDtypeStruct((K * NC * NP, FH), jnp.float32),
        compiler_params=pltpu.CompilerParams(needs_layout_passes=False),
        scratch_types=[
            pltpu.VMEM((SB, EB), jnp.int32),
            pltpu.VMEM((SB, EB), jnp.int32),
            pltpu.VMEM((SB * EB // 16, 16), jnp.float32),
            pltpu.VMEM((EB, FH), jnp.float32),
            pltpu.VMEM((EB, FH), jnp.float32),
            pltpu.VMEM((EB // 2, FH), jnp.float32),
            pltpu.VMEM_SHARED((NP, FH), jnp.float32),
            pltpu.SemaphoreType.DMA,
            pltpu.SemaphoreType.DMA,
            pltpu.SemaphoreType.DMA,
            pltpu.SemaphoreType.DMA,
        ],
    )
    def prop_kernel(h_hbm, row_hbm, col_hbm, ew2_hbm, out_hbm,
                    ridx_v, cidx_v, ew_v, buf_a, buf_b, zbuf, acc,
                    sem_a, sem_b, sem_c, sem_d):
        c = lax.axis_index("c")
        s = lax.axis_index("s")
        lane = lax.iota(jnp.int32, 16)
        zero16 = jnp.zeros((16,), jnp.float32)

        def zfill(e, carry):
            ef = jnp.full((16,), e, jnp.int32)
            for j in range(FH // 16):
                plsc.store_scatter(zbuf, [ef, j * 16 + lane], zero16)
            return carry

        lax.fori_loop(0, EB // 2, zfill, 0)

        def stack_body(k, carry):
            def zero_body(q, carry2):
                off = pl.multiple_of(s * rpt + q * (EB // 2), 8)
                pltpu.sync_copy(zbuf, acc.at[pl.ds(off, EB // 2)])
                return carry2

            lax.fori_loop(0, rpt // (EB // 2), zero_body, 0)
            plsc.subcore_barrier()

            base_row = pl.multiple_of((k * NC + c) * NP, NP)
            h_k = h_hbm.at[pl.ds(base_row, NP)]

            def chunk_body(sb, carry2):
                erow = pl.multiple_of(s * nb + sb * SB, 8)
                pltpu.sync_copy(row_hbm.at[pl.ds(erow, SB)], ridx_v)
                pltpu.sync_copy(col_hbm.at[pl.ds(erow, SB)], cidx_v)
                pltpu.sync_copy(
                    ew2_hbm.at[pl.ds(pl.multiple_of(erow * (EB // 16), 8),
                                     SB * EB // 16)], ew_v)

                def scale(buf, b):
                    ewrow = b * (EB // 16)
                    for e in range(EB):
                        w = plsc.load_gather(
                            ew_v, [jnp.full((16,), e // 16, jnp.int32) + ewrow,
                                   jnp.full((16,), e % 16, jnp.int32)])
                        for j in range(FH // 16):
                            sl = pl.ds(j * 16, 16)
                            buf[e, sl] = buf[e, sl] * w

                def pair_body(p, carry3):
                    b0 = p * 2
                    b1 = b0 + 1
                    cp_a = pltpu.async_copy(h_k.at[ridx_v.at[b0]], buf_a, sem_a)
                    cp_b = pltpu.async_copy(h_k.at[ridx_v.at[b1]], buf_b, sem_b)
                    cp_a.wait()
                    scale(buf_a, b0)
                    sp_a = pltpu.async_copy(
                        buf_a, acc.at[cidx_v.at[b0]], sem_c, add=True)
                    cp_b.wait()
                    scale(buf_b, b1)
                    sp_b = pltpu.async_copy(
                        buf_b, acc.at[cidx_v.at[b1]], sem_d, add=True)
                    sp_a.wait()
                    sp_b.wait()
                    return carry3

                lax.fori_loop(0, SB // 2, pair_body, 0)
                return carry2

            lax.fori_loop(0, nb // SB, chunk_body, 0)
            plsc.subcore_barrier()
            pltpu.sync_copy(
                acc.at[pl.ds(pl.multiple_of(s * rpt, rpt), rpt)],
                out_hbm.at[pl.ds(pl.multiple_of(base_row + s * rpt, rpt), rpt)],
            )
            return carry

        lax.fori_loop(0, K, stack_body, 0)

    return prop_kernel


def _dinv(deg_parts):
    deg = jnp.sum(deg_parts, axis=0)
    return jnp.where(deg > 0, lax.rsqrt(jnp.maximum(deg, 1e-12)), 0.0)[:, None]


def _prep_body(deg_ref, x_ref, iw_ref, rw_ref, b_ref, h_ref, rootb_ref):
    dinv = _dinv(deg_ref[...])
    xb = x_ref[...]
    for k in range(K):
        rootb_ref[k] = (
            jnp.dot(xb, rw_ref[k], preferred_element_type=jnp.float32) + b_ref[k]
        )
        h = jnp.dot(xb, iw_ref[k], preferred_element_type=jnp.float32) * dinv
        h_ref[k, 0] = h[:, :FH]
        h_ref[k, 1] = h[:, FH:]


def _mid_body(deg_ref, acc_ref, rootb_ref, w_ref, h_ref):
    dinv = _dinv(deg_ref[...])
    for k in range(K):
        agg = jnp.concatenate([acc_ref[k, 0], acc_ref[k, 1]], axis=-1) * dinv
        out1 = jnp.maximum(agg + rootb_ref[k], 0.0)
        h = jnp.dot(out1, w_ref[k], preferred_element_type=jnp.float32) * dinv
        h_ref[k, 0] = h[:, :FH]
        h_ref[k, 1] = h[:, FH:]


def _final_body(deg_ref, acc_ref, rootb_ref, out_ref):
    dinv = _dinv(deg_ref[...])
    acc_sum = jnp.zeros(out_ref.shape, jnp.float32)
    for k in range(K):
        agg = jnp.concatenate([acc_ref[k, 0], acc_ref[k, 1]], axis=-1) * dinv
        acc_sum = acc_sum + jnp.maximum(agg + rootb_ref[k], 0.0)
    out_ref[...] = jnp.maximum(acc_sum * (1.0 / K), 0.0)


def kernel(x, edge_index, edge_weight, init_weight, weight, root_weight, bias):
    N = x.shape[0]
    E = edge_index.shape[1]
    NP = _round_up(N, NS * 128)
    EP = _round_up(E, NS * EB * 16)

    row = edge_index[0]
    col = edge_index[1]
    epad = EP - E
    row_p = jnp.pad(row, (0, epad))
    col_p = jnp.pad(col, (0, epad))
    ew_p = jnp.pad(edge_weight, (0, epad))
    x_p = jnp.pad(x, ((0, NP - N), (0, 0)))

    grid = NP // BM
    deg_spec = pl.BlockSpec((NC * NS, BM), lambda m: (0, m))
    mat_spec = pl.BlockSpec((BM, F), lambda m: (m, 0))
    h_spec = pl.BlockSpec((K, NC, BM, FH), lambda m: (0, 0, m, 0))
    rootb_spec = pl.BlockSpec((K, BM, F), lambda m: (0, m, 0))
    w3_spec = pl.BlockSpec((K, F, F), lambda m: (0, 0, 0))
    b_spec = pl.BlockSpec((K, 1, F), lambda m: (0, 0, 0))

    deg_parts = _make_deg(NP, EP)(col_p, ew_p).reshape(NC * NS, NP)
    ew2 = ew_p.reshape(EP // 16, 16)
    row2 = row_p.reshape(EP // EB, EB)
    col2 = col_p.reshape(EP // EB, EB)

    H0, rootb = pl.pallas_call(
        _prep_body,
        grid=(grid,),
        in_specs=[deg_spec, mat_spec, w3_spec, w3_spec, b_spec],
        out_specs=[h_spec, rootb_spec],
        out_shape=[
            jax.ShapeDtypeStruct((K, NC, NP, FH), jnp.float32),
            jax.ShapeDtypeStruct((K, NP, F), jnp.float32),
        ],
    )(deg_parts, x_p, init_weight, root_weight[0], bias[0])

    prop = _make_prop(NP, EP)

    acc1 = prop(H0.reshape(K * NC * NP, FH), row2, col2, ew2)

    H1 = pl.pallas_call(
        _mid_body,
        grid=(grid,),
        in_specs=[deg_spec, h_spec, rootb_spec, w3_spec],
        out_specs=h_spec,
        out_shape=jax.ShapeDtypeStruct((K, NC, NP, FH), jnp.float32),
    )(deg_parts, acc1.reshape(K, NC, NP, FH), rootb, weight[0])

    acc2 = prop(H1.reshape(K * NC * NP, FH), row2, col2, ew2)

    out = pl.pallas_call(
        _final_body,
        grid=(grid,),
        in_specs=[deg_spec, h_spec, rootb_spec],
        out_specs=pl.BlockSpec((BM, F), lambda m: (m, 0)),
        out_shape=jax.ShapeDtypeStruct((NP, F), jnp.float32),
    )(deg_parts, acc2.reshape(K, NC, NP, FH), rootb)

    return out[:N]


# speedup vs baseline: 1.2656x; 1.2656x over previous
import functools

import jax
import jax.numpy as jnp
from jax import lax
from jax.experimental import pallas as pl
from jax.experimental.pallas import tpu as pltpu
from jax.experimental.pallas import tpu_sc as plsc

F = 256
FH = 128
K = 2
NC = 2
NS = 16
EB = 64
SB = 8
BM = 512


def _round_up(a, b):
    return (a + b - 1) // b * b


def _make_deg(NP, EP):
    mesh = plsc.VectorSubcoreMesh(core_axis_name="c", subcore_axis_name="s")
    epw = EP // (NC * NS)
    nb = epw // EB

    @functools.partial(
        pl.kernel,
        mesh=mesh,
        out_type=jax.ShapeDtypeStruct((NC * NS, NP // 128, 128), jnp.float32),
        compiler_params=pltpu.CompilerParams(needs_layout_passes=False),
        scratch_types=[
            pltpu.VMEM((EB,), jnp.int32),
            pltpu.VMEM((EB,), jnp.float32),
            pltpu.VMEM((NP // 128, 128), jnp.float32),
        ],
    )
    def deg_kernel(col_hbm, ew_hbm, out_hbm, cidx_v, ew_v, deg_v):
        c = lax.axis_index("c")
        s = lax.axis_index("s")
        wid = s * NC + c
        lane = lax.iota(jnp.int32, 16)
        zero16 = jnp.zeros((16,), jnp.float32)

        def zero_body(r, carry):
            rf = jnp.full((16,), r, jnp.int32)
            for j in range(8):
                plsc.store_scatter(deg_v, [rf, j * 16 + lane], zero16)
            return carry

        lax.fori_loop(0, NP // 128, zero_body, 0)

        base0 = wid * epw

        def batch_body(b, carry):
            eb = pl.multiple_of(base0 + b * EB, EB)
            pltpu.sync_copy(col_hbm.at[pl.ds(eb, EB)], cidx_v)
            pltpu.sync_copy(ew_hbm.at[pl.ds(eb, EB)], ew_v)
            for j in range(EB // 16):
                idx = cidx_v[pl.ds(j * 16, 16)]
                w = ew_v[pl.ds(j * 16, 16)]
                plsc.addupdate_scatter(
                    deg_v, [lax.shift_right_logical(idx, 7), idx & 127], w)
            return carry

        lax.fori_loop(0, nb, batch_body, 0)
        pltpu.sync_copy(deg_v, out_hbm.at[wid])

    return deg_kernel


def _make_prop(NP, EP):
    mesh = plsc.VectorSubcoreMesh(core_axis_name="c", subcore_axis_name="s")
    ept = EP // NS
    nb = ept // EB
    rpt = NP // NS

    @functools.partial(
        pl.kernel,
        mesh=mesh,
        out_type=jax.ShapeDtypeStruct((K * NC * NP, FH), jnp.float32),
        compiler_params=pltpu.CompilerParams(
            needs_layout_passes=False, use_tc_tiling_on_sc=False),
        scratch_types=[
            pltpu.VMEM((SB, EB), jnp.int32),
            pltpu.VMEM((SB, EB), jnp.int32),
            pltpu.VMEM((SB * EB // 16, 16), jnp.float32),
            pltpu.VMEM((EB, FH // 2), jnp.int32),
            pltpu.VMEM((EB, FH // 2), jnp.int32),
            pltpu.VMEM((EB, FH), jnp.float32),
            pltpu.VMEM((EB, FH), jnp.float32),
            pltpu.VMEM((EB, FH), jnp.float32),
            pltpu.VMEM_SHARED((NP, FH), jnp.float32),
            pltpu.SemaphoreType.DMA,
            pltpu.SemaphoreType.DMA,
            pltpu.SemaphoreType.DMA,
            pltpu.SemaphoreType.DMA,
        ],
    )
    def prop_kernel(h_hbm, row_hbm, col_hbm, ew2_hbm, out_hbm,
                    ridx_v, cidx_v, ew_v, bg_a, bg_b, bs_a, bs_b, zbuf, acc,
                    sem_a, sem_b, sem_c, sem_d):
        c = lax.axis_index("c")
        s = lax.axis_index("s")
        lane = lax.iota(jnp.int32, 16)
        zero16 = jnp.zeros((16,), jnp.float32)

        def zfill(e, carry):
            ef = jnp.full((16,), e, jnp.int32)
            for j in range(FH // 16):
                plsc.store_scatter(zbuf, [ef, j * 16 + lane], zero16)
            return carry

        lax.fori_loop(0, EB, zfill, 0)

        def stack_body(k, carry):
            def zero_body(q, carry2):
                off = pl.multiple_of(s * rpt + q * EB, 8)
                pltpu.sync_copy(zbuf, acc.at[pl.ds(off, EB)])
                return carry2

            lax.fori_loop(0, rpt // EB, zero_body, 0)
            plsc.subcore_barrier()

            base_row = pl.multiple_of((k * NC + c) * NP, NP)
            h_k = h_hbm.at[pl.ds(base_row, NP)]

            def chunk_body(sb, carry2):
                erow = pl.multiple_of(s * nb + sb * SB, 8)
                pltpu.sync_copy(row_hbm.at[pl.ds(erow, SB)], ridx_v)
                pltpu.sync_copy(col_hbm.at[pl.ds(erow, SB)], cidx_v)
                pltpu.sync_copy(
                    ew2_hbm.at[pl.ds(pl.multiple_of(erow * (EB // 16), 8),
                                     SB * EB // 16)], ew_v)

                def scale(bg, bs, b):
                    ewrow = b * (EB // 16)
                    for e in range(EB):
                        w = plsc.load_gather(
                            ew_v, [jnp.full((16,), e // 16, jnp.int32) + ewrow,
                                   jnp.full((16,), e % 16, jnp.int32)])
                        for j in range(4):
                            v = bg[e, pl.ds(j * 16, 16)]
                            vb = plsc.bitcast(v, jnp.bfloat16)
                            lo, hi = plsc.unpack(
                                vb, format=plsc.PackFormat.INTERLEAVED)
                            bs[e, pl.ds(j * 16, 16)] = lo * w
                            bs[e, pl.ds(64 + j * 16, 16)] = hi * w

                def pair_body(p, carry3):
                    b0 = p * 2
                    b1 = b0 + 1
                    cp_a = pltpu.async_copy(h_k.at[ridx_v.at[b0]], bg_a, sem_a)
                    cp_b = pltpu.async_copy(h_k.at[ridx_v.at[b1]], bg_b, sem_b)
                    cp_a.wait()
                    scale(bg_a, bs_a, b0)
                    sp_a = pltpu.async_copy(
                        bs_a, acc.at[cidx_v.at[b0]], sem_c, add=True)
                    cp_b.wait()
                    scale(bg_b, bs_b, b1)
                    sp_b = pltpu.async_copy(
                        bs_b, acc.at[cidx_v.at[b1]], sem_d, add=True)
                    sp_a.wait()
                    sp_b.wait()
                    return carry3

                lax.fori_loop(0, SB // 2, pair_body, 0)
                return carry2

            lax.fori_loop(0, nb // SB, chunk_body, 0)
            plsc.subcore_barrier()
            pltpu.sync_copy(
                acc.at[pl.ds(pl.multiple_of(s * rpt, rpt), rpt)],
                out_hbm.at[pl.ds(pl.multiple_of(base_row + s * rpt, rpt), rpt)],
            )
            return carry

        lax.fori_loop(0, K, stack_body, 0)

    return prop_kernel


def _dinv(deg_parts):
    deg = jnp.sum(deg_parts, axis=0)
    return jnp.where(deg > 0, lax.rsqrt(jnp.maximum(deg, 1e-12)), 0.0)[:, None]


def _agg(acc_ref, k, dinv):
    return jnp.concatenate([acc_ref[k, 0], acc_ref[k, 1]], axis=-1) * dinv


def _bf16_bits(x):
    b = lax.bitcast_convert_type(x, jnp.int32)
    r = b + 0x7FFF + (lax.shift_right_logical(b, 16) & 1)
    return lax.shift_right_logical(r, 16)


def _pack_h(h_ref, h, k):
    for c in range(NC):
        lo = _bf16_bits(h[:, c * FH:c * FH + 64])
        hi = _bf16_bits(h[:, c * FH + 64:(c + 1) * FH])
        h_ref[k, c] = lax.shift_left(hi, 16) | lo


def _prep_body(deg_ref, x_ref, iw_ref, rw_ref, b_ref, h_ref, rootb_ref):
    dinv = _dinv(deg_ref[...])
    xb = x_ref[...]
    for k in range(K):
        rootb_ref[k] = (
            jnp.dot(xb, rw_ref[k], preferred_element_type=jnp.float32) + b_ref[k]
        )
        h = jnp.dot(xb, iw_ref[k], preferred_element_type=jnp.float32) * dinv
        _pack_h(h_ref, h, k)


def _mid_body(deg_ref, acc_ref, rootb_ref, w_ref, h_ref):
    dinv = _dinv(deg_ref[...])
    for k in range(K):
        out1 = jnp.maximum(_agg(acc_ref, k, dinv) + rootb_ref[k], 0.0)
        h = jnp.dot(out1, w_ref[k], preferred_element_type=jnp.float32) * dinv
        _pack_h(h_ref, h, k)


def _final_body(deg_ref, acc_ref, rootb_ref, out_ref):
    dinv = _dinv(deg_ref[...])
    acc_sum = jnp.zeros(out_ref.shape, jnp.float32)
    for k in range(K):
        acc_sum = acc_sum + jnp.maximum(_agg(acc_ref, k, dinv) + rootb_ref[k], 0.0)
    out_ref[...] = jnp.maximum(acc_sum * (1.0 / K), 0.0)


def kernel(x, edge_index, edge_weight, init_weight, weight, root_weight, bias):
    N = x.shape[0]
    E = edge_index.shape[1]
    NP = _round_up(N, NS * 128)
    EP = _round_up(E, NS * EB * 16)

    row = edge_index[0]
    col = edge_index[1]
    epad = EP - E
    row_p = jnp.pad(row, (0, epad))
    col_p = jnp.pad(col, (0, epad))
    ew_p = jnp.pad(edge_weight, (0, epad))
    x_p = jnp.pad(x, ((0, NP - N), (0, 0)))

    grid = NP // BM
    deg_spec = pl.BlockSpec((NC * NS, BM), lambda m: (0, m))
    mat_spec = pl.BlockSpec((BM, F), lambda m: (m, 0))
    h_spec = pl.BlockSpec((K, NC, BM, 64), lambda m: (0, 0, m, 0))
    acc_spec = pl.BlockSpec((K, NC, BM, FH), lambda m: (0, 0, m, 0))
    rootb_spec = pl.BlockSpec((K, BM, F), lambda m: (0, m, 0))
    w3_spec = pl.BlockSpec((K, F, F), lambda m: (0, 0, 0))
    b_spec = pl.BlockSpec((K, 1, F), lambda m: (0, 0, 0))

    deg_parts = _make_deg(NP, EP)(col_p, ew_p).reshape(NC * NS, NP)
    ew2 = ew_p.reshape(EP // 16, 16)
    row2 = row_p.reshape(EP // EB, EB)
    col2 = col_p.reshape(EP // EB, EB)

    H0, rootb = pl.pallas_call(
        _prep_body,
        grid=(grid,),
        in_specs=[deg_spec, mat_spec, w3_spec, w3_spec, b_spec],
        out_specs=[h_spec, rootb_spec],
        out_shape=[
            jax.ShapeDtypeStruct((K, NC, NP, 64), jnp.int32),
            jax.ShapeDtypeStruct((K, NP, F), jnp.float32),
        ],
    )(deg_parts, x_p, init_weight, root_weight[0], bias[0])

    prop = _make_prop(NP, EP)

    acc1 = prop(H0.reshape(K * NC * NP, 64), row2, col2, ew2)

    H1 = pl.pallas_call(
        _mid_body,
        grid=(grid,),
        in_specs=[deg_spec, acc_spec, rootb_spec, w3_spec],
        out_specs=h_spec,
        out_shape=jax.ShapeDtypeStruct((K, NC, NP, 64), jnp.int32),
    )(deg_parts, acc1.reshape(K, NC, NP, FH), rootb, weight[0])

    acc2 = prop(H1.reshape(K * NC * NP, 64), row2, col2, ew2)

    out = pl.pallas_call(
        _final_body,
        grid=(grid,),
        in_specs=[deg_spec, acc_spec, rootb_spec],
        out_specs=pl.BlockSpec((BM, F), lambda m: (m, 0)),
        out_shape=jax.ShapeDtypeStruct((NP, F), jnp.float32),
    )(deg_parts, acc2.reshape(K, NC, NP, FH), rootb)

    return out[:N]


# speedup vs baseline: 1.3330x; 1.0533x over previous
import functools

import jax
import jax.numpy as jnp
from jax import lax
from jax.experimental import pallas as pl
from jax.experimental.pallas import tpu as pltpu
from jax.experimental.pallas import tpu_sc as plsc

F = 256
FH = 128
K = 2
NC = 2
NS = 16
EB = 64
SB = 8
BM = 512


def _round_up(a, b):
    return (a + b - 1) // b * b


def _make_deg(NP, EP):
    mesh = plsc.VectorSubcoreMesh(core_axis_name="c", subcore_axis_name="s")
    epw = EP // (NC * NS)
    nb = epw // EB

    @functools.partial(
        pl.kernel,
        mesh=mesh,
        out_type=jax.ShapeDtypeStruct((NC * NS, NP // 128, 128), jnp.float32),
        compiler_params=pltpu.CompilerParams(needs_layout_passes=False),
        scratch_types=[
            pltpu.VMEM((EB,), jnp.int32),
            pltpu.VMEM((EB,), jnp.float32),
            pltpu.VMEM((NP // 128, 128), jnp.float32),
        ],
    )
    def deg_kernel(col_hbm, ew_hbm, out_hbm, cidx_v, ew_v, deg_v):
        c = lax.axis_index("c")
        s = lax.axis_index("s")
        wid = s * NC + c
        lane = lax.iota(jnp.int32, 16)
        zero16 = jnp.zeros((16,), jnp.float32)

        def zero_body(r, carry):
            rf = jnp.full((16,), r, jnp.int32)
            for j in range(8):
                plsc.store_scatter(deg_v, [rf, j * 16 + lane], zero16)
            return carry

        lax.fori_loop(0, NP // 128, zero_body, 0)

        base0 = wid * epw

        def batch_body(b, carry):
            eb = pl.multiple_of(base0 + b * EB, EB)
            pltpu.sync_copy(col_hbm.at[pl.ds(eb, EB)], cidx_v)
            pltpu.sync_copy(ew_hbm.at[pl.ds(eb, EB)], ew_v)
            for j in range(EB // 16):
                idx = cidx_v[pl.ds(j * 16, 16)]
                w = ew_v[pl.ds(j * 16, 16)]
                plsc.addupdate_scatter(
                    deg_v, [lax.shift_right_logical(idx, 7), idx & 127], w)
            return carry

        lax.fori_loop(0, nb, batch_body, 0)
        pltpu.sync_copy(deg_v, out_hbm.at[wid])

    return deg_kernel


def _make_prop(NP, EP):
    mesh = plsc.VectorSubcoreMesh(core_axis_name="c", subcore_axis_name="s")
    ept = EP // NS
    nb = ept // EB
    rpt = NP // NS

    @functools.partial(
        pl.kernel,
        mesh=mesh,
        out_type=jax.ShapeDtypeStruct((K * NC * NP, FH), jnp.float32),
        compiler_params=pltpu.CompilerParams(
            needs_layout_passes=False, use_tc_tiling_on_sc=False),
        scratch_types=[
            pltpu.VMEM((SB, EB), jnp.int32),
            pltpu.VMEM((SB, EB), jnp.int32),
            pltpu.VMEM((SB * EB // 16, 16), jnp.float32),
            pltpu.VMEM((EB, FH // 2), jnp.int32),
            pltpu.VMEM((EB, FH // 2), jnp.int32),
            pltpu.VMEM((EB, FH), jnp.float32),
            pltpu.VMEM((EB, FH), jnp.float32),
            pltpu.VMEM((EB, FH), jnp.float32),
            pltpu.VMEM_SHARED((NP, FH), jnp.float32),
            pltpu.SemaphoreType.DMA,
            pltpu.SemaphoreType.DMA,
            pltpu.SemaphoreType.DMA,
            pltpu.SemaphoreType.DMA,
        ],
    )
    def prop_kernel(h_hbm, row_hbm, col_hbm, ew2_hbm, out_hbm,
                    ridx_v, cidx_v, ew_v, bg_a, bg_b, bs_a, bs_b, zbuf, acc,
                    sem_a, sem_b, sem_c, sem_d):
        c = lax.axis_index("c")
        s = lax.axis_index("s")
        lane = lax.iota(jnp.int32, 16)
        zero16 = jnp.zeros((16,), jnp.float32)

        def zfill(e, carry):
            ef = jnp.full((16,), e, jnp.int32)
            for j in range(FH // 16):
                plsc.store_scatter(zbuf, [ef, j * 16 + lane], zero16)
            return carry

        lax.fori_loop(0, EB, zfill, 0)

        def stack_body(k, carry):
            def zero_body(q, carry2):
                off = pl.multiple_of(s * rpt + q * EB, 8)
                pltpu.sync_copy(zbuf, acc.at[pl.ds(off, EB)])
                return carry2

            lax.fori_loop(0, rpt // EB, zero_body, 0)
            plsc.subcore_barrier()

            base_row = pl.multiple_of((k * NC + c) * NP, NP)
            h_k = h_hbm.at[pl.ds(base_row, NP)]

            def chunk_body(sb, carry2):
                erow = pl.multiple_of(s * nb + sb * SB, 8)
                pltpu.sync_copy(row_hbm.at[pl.ds(erow, SB)], ridx_v)
                pltpu.sync_copy(col_hbm.at[pl.ds(erow, SB)], cidx_v)
                pltpu.sync_copy(
                    ew2_hbm.at[pl.ds(pl.multiple_of(erow * (EB // 16), 8),
                                     SB * EB // 16)], ew_v)

                def scale(bg, bs, b):
                    ewrow = b * (EB // 16)
                    for e in range(EB):
                        w = plsc.load_gather(
                            ew_v, [jnp.full((16,), e // 16, jnp.int32) + ewrow,
                                   jnp.full((16,), e % 16, jnp.int32)])
                        for j in range(4):
                            v = bg[e, pl.ds(j * 16, 16)]
                            vb = plsc.bitcast(v, jnp.bfloat16)
                            lo, hi = plsc.unpack(
                                vb, format=plsc.PackFormat.INTERLEAVED)
                            bs[e, pl.ds(j * 16, 16)] = lo * w
                            bs[e, pl.ds(64 + j * 16, 16)] = hi * w

                def drain(sem):
                    pltpu.make_async_copy(
                        out_hbm.at[pl.ds(0, EB)], bs_a, sem).wait()

                def pair_body(p, carry3):
                    b0 = p * 2
                    b1 = b0 + 1
                    cp_a = pltpu.async_copy(h_k.at[ridx_v.at[b0]], bg_a, sem_a)
                    cp_b = pltpu.async_copy(h_k.at[ridx_v.at[b1]], bg_b, sem_b)
                    cp_a.wait()

                    @pl.when(p > 0)
                    def _():
                        drain(sem_c)

                    scale(bg_a, bs_a, b0)
                    pltpu.async_copy(bs_a, acc.at[cidx_v.at[b0]], sem_c, add=True)
                    cp_b.wait()

                    @pl.when(p > 0)
                    def _():
                        drain(sem_d)

                    scale(bg_b, bs_b, b1)
                    pltpu.async_copy(bs_b, acc.at[cidx_v.at[b1]], sem_d, add=True)
                    return carry3

                lax.fori_loop(0, SB // 2, pair_body, 0)
                drain(sem_c)
                drain(sem_d)
                return carry2

            lax.fori_loop(0, nb // SB, chunk_body, 0)
            plsc.subcore_barrier()
            pltpu.sync_copy(
                acc.at[pl.ds(pl.multiple_of(s * rpt, rpt), rpt)],
                out_hbm.at[pl.ds(pl.multiple_of(base_row + s * rpt, rpt), rpt)],
            )
            return carry

        lax.fori_loop(0, K, stack_body, 0)

    return prop_kernel


def _dinv(deg_parts):
    deg = jnp.sum(deg_parts, axis=0)
    return jnp.where(deg > 0, lax.rsqrt(jnp.maximum(deg, 1e-12)), 0.0)[:, None]


def _agg(acc_ref, k, dinv):
    return jnp.concatenate([acc_ref[k, 0], acc_ref[k, 1]], axis=-1) * dinv


def _bf16_bits(x):
    b = lax.bitcast_convert_type(x, jnp.int32)
    r = b + 0x7FFF + (lax.shift_right_logical(b, 16) & 1)
    return lax.shift_right_logical(r, 16)


def _pack_h(h_ref, h, k):
    for c in range(NC):
        lo = _bf16_bits(h[:, c * FH:c * FH + 64])
        hi = _bf16_bits(h[:, c * FH + 64:(c + 1) * FH])
        h_ref[k, c] = lax.shift_left(hi, 16) | lo


def _prep_body(deg_ref, x_ref, iw_ref, rw_ref, b_ref, h_ref, rootb_ref):
    dinv = _dinv(deg_ref[...])
    xb = x_ref[...]
    for k in range(K):
        rootb_ref[k] = (
            jnp.dot(xb, rw_ref[k], preferred_element_type=jnp.float32) + b_ref[k]
        )
        h = jnp.dot(xb, iw_ref[k], preferred_element_type=jnp.float32) * dinv
        _pack_h(h_ref, h, k)


def _mid_body(deg_ref, acc_ref, rootb_ref, w_ref, h_ref):
    dinv = _dinv(deg_ref[...])
    for k in range(K):
        out1 = jnp.maximum(_agg(acc_ref, k, dinv) + rootb_ref[k], 0.0)
        h = jnp.dot(out1, w_ref[k], preferred_element_type=jnp.float32) * dinv
        _pack_h(h_ref, h, k)


def _final_body(deg_ref, acc_ref, rootb_ref, out_ref):
    dinv = _dinv(deg_ref[...])
    acc_sum = jnp.zeros(out_ref.shape, jnp.float32)
    for k in range(K):
        acc_sum = acc_sum + jnp.maximum(_agg(acc_ref, k, dinv) + rootb_ref[k], 0.0)
    out_ref[...] = jnp.maximum(acc_sum * (1.0 / K), 0.0)


def kernel(x, edge_index, edge_weight, init_weight, weight, root_weight, bias):
    N = x.shape[0]
    E = edge_index.shape[1]
    NP = _round_up(N, NS * 128)
    EP = _round_up(E, NS * EB * 16)

    row = edge_index[0]
    col = edge_index[1]
    epad = EP - E
    row_p = jnp.pad(row, (0, epad))
    col_p = jnp.pad(col, (0, epad))
    ew_p = jnp.pad(edge_weight, (0, epad))
    x_p = jnp.pad(x, ((0, NP - N), (0, 0)))

    grid = NP // BM
    deg_spec = pl.BlockSpec((NC * NS, BM), lambda m: (0, m))
    mat_spec = pl.BlockSpec((BM, F), lambda m: (m, 0))
    h_spec = pl.BlockSpec((K, NC, BM, 64), lambda m: (0, 0, m, 0))
    acc_spec = pl.BlockSpec((K, NC, BM, FH), lambda m: (0, 0, m, 0))
    rootb_spec = pl.BlockSpec((K, BM, F), lambda m: (0, m, 0))
    w3_spec = pl.BlockSpec((K, F, F), lambda m: (0, 0, 0))
    b_spec = pl.BlockSpec((K, 1, F), lambda m: (0, 0, 0))

    deg_parts = _make_deg(NP, EP)(col_p, ew_p).reshape(NC * NS, NP)
    ew2 = ew_p.reshape(EP // 16, 16)
    row2 = row_p.reshape(EP // EB, EB)
    col2 = col_p.reshape(EP // EB, EB)

    H0, rootb = pl.pallas_call(
        _prep_body,
        grid=(grid,),
        in_specs=[deg_spec, mat_spec, w3_spec, w3_spec, b_spec],
        out_specs=[h_spec, rootb_spec],
        out_shape=[
            jax.ShapeDtypeStruct((K, NC, NP, 64), jnp.int32),
            jax.ShapeDtypeStruct((K, NP, F), jnp.float32),
        ],
    )(deg_parts, x_p, init_weight, root_weight[0], bias[0])

    prop = _make_prop(NP, EP)

    acc1 = prop(H0.reshape(K * NC * NP, 64), row2, col2, ew2)

    H1 = pl.pallas_call(
        _mid_body,
        grid=(grid,),
        in_specs=[deg_spec, acc_spec, rootb_spec, w3_spec],
        out_specs=h_spec,
        out_shape=jax.ShapeDtypeStruct((K, NC, NP, 64), jnp.int32),
    )(deg_parts, acc1.reshape(K, NC, NP, FH), rootb, weight[0])

    acc2 = prop(H1.reshape(K * NC * NP, 64), row2, col2, ew2)

    out = pl.pallas_call(
        _final_body,
        grid=(grid,),
        in_specs=[deg_spec, acc_spec, rootb_spec],
        out_specs=pl.BlockSpec((BM, F), lambda m: (m, 0)),
        out_shape=jax.ShapeDtypeStruct((NP, F), jnp.float32),
    )(deg_parts, acc2.reshape(K, NC, NP, FH), rootb)

    return out[:N]
